# Initial kernel scaffold; baseline (speedup 1.0000x reference)
import jax
import jax.numpy as jnp
from jax.experimental import pallas as pl


def kernel(node_features, edge_index, edge_features, Wl, bl, Wr, br, We, be, attn_vec, Wv, bv, Wo, bo):
    raise NotImplementedError("write your pallas kernel here")


# speedup vs baseline: 6.4204x; 6.4204x over previous
import functools

import jax
import jax.numpy as jnp
from jax import lax
from jax.experimental import pallas as pl
from jax.experimental.pallas import tpu as pltpu
from jax.experimental.pallas import tpu_sc as plsc

N = 10000
E = 320000
DF = 128
H = 8
HD = 16
NW = 32
EPW = E // NW
C = 80
NCH = EPW // C
NPAD = 10240
NPT = NPAD // 16


def _tc_proj(x, Wl, bl, Wr, br, Wv, bv):
    BN = 1000

    def body(x_ref, wl, bl_, wr, br_, wv, bv_, lo, ro, vlo, vhi):
        xb = x_ref[...]
        lo[...] = jnp.dot(xb, wl[...], preferred_element_type=jnp.float32) + bl_[...]
        ro[...] = jnp.dot(xb, wr[...], preferred_element_type=jnp.float32) + br_[...]
        v = jnp.dot(xb, wv[...], preferred_element_type=jnp.float32) + bv_[...]
        vlo[...] = v[:, :64]
        vhi[...] = v[:, 64:]

    w_spec = pl.BlockSpec((DF, DF), lambda i: (0, 0))
    b_spec = pl.BlockSpec((1, DF), lambda i: (0, 0))
    o_spec = pl.BlockSpec((BN, DF), lambda i: (i, 0))
    h_spec = pl.BlockSpec((BN, 64), lambda i: (i, 0))
    return pl.pallas_call(
        body,
        grid=(N // BN,),
        in_specs=[o_spec, w_spec, b_spec, w_spec, b_spec, w_spec, b_spec],
        out_specs=[o_spec, o_spec, h_spec, h_spec],
        out_shape=[jax.ShapeDtypeStruct((N, DF), jnp.float32)] * 2
        + [jax.ShapeDtypeStruct((N, 64), jnp.float32)] * 2,
    )(x, Wl, bl.reshape(1, DF), Wr, br.reshape(1, DF), Wv, bv.reshape(1, DF))


def _sc_combine(left, right, src_arr, tgt_arr):
    mesh = plsc.VectorSubcoreMesh(core_axis_name="c", subcore_axis_name="s")

    @functools.partial(
        pl.kernel,
        out_type=jax.ShapeDtypeStruct((E, DF), jnp.float32),
        mesh=mesh,
        scratch_types=[
            pltpu.VMEM((C,), jnp.int32),
            pltpu.VMEM((C,), jnp.int32),
            pltpu.VMEM((C, DF), jnp.float32),
            pltpu.VMEM((C, DF), jnp.float32),
            pltpu.SemaphoreType.DMA,
            pltpu.SemaphoreType.DMA,
        ],
    )
    def k(left_hbm, right_hbm, src_hbm, tgt_hbm, out_hbm,
          tgt_v, src_v, lbuf, rbuf, sem1, sem2):
        wid = lax.axis_index("s") * 2 + lax.axis_index("c")

        def chunk(g, carry):
            base = wid * EPW + g * C
            pltpu.sync_copy(tgt_hbm.at[pl.ds(base, C)], tgt_v)
            pltpu.sync_copy(src_hbm.at[pl.ds(base, C)], src_v)
            cl = pltpu.async_copy(left_hbm.at[tgt_v], lbuf, sem1)
            cr = pltpu.async_copy(right_hbm.at[src_v], rbuf, sem2)
            cl.wait()
            cr.wait()

            def row(i, c2):
                for j in range(DF // 16):
                    sl = pl.ds(j * 16, 16)
                    lbuf[i, sl] = lbuf[i, sl] + rbuf[i, sl]
                return c2

            lax.fori_loop(0, C, row, 0)
            pltpu.sync_copy(lbuf, out_hbm.at[pl.ds(base, C)])
            return carry

        lax.fori_loop(0, NCH, chunk, 0)

    return k(left, right, src_arr, tgt_arr)


def _tc_scores(combined, ef, A16, We16, be16):
    BE = 2000

    def body(c_ref, e_ref, a_ref, w_ref, b_ref, out):
        c = c_ref[...]
        act = jnp.where(c >= 0, c, 0.2 * c)
        s = (jnp.dot(act, a_ref[...], preferred_element_type=jnp.float32)
             + jnp.dot(e_ref[...], w_ref[...], preferred_element_type=jnp.float32)
             + b_ref[...])
        out[...] = jnp.exp(s)

    return pl.pallas_call(
        body,
        grid=(E // BE,),
        in_specs=[
            pl.BlockSpec((BE, DF), lambda i: (i, 0)),
            pl.BlockSpec((BE, 16), lambda i: (i, 0)),
            pl.BlockSpec((DF, 16), lambda i: (0, 0)),
            pl.BlockSpec((16, 16), lambda i: (0, 0)),
            pl.BlockSpec((1, 16), lambda i: (0, 0)),
        ],
        out_specs=pl.BlockSpec((BE, 16), lambda i: (i, 0)),
        out_shape=jax.ShapeDtypeStruct((E, 16), jnp.float32),
    )(combined, ef, A16, We16, be16)


def _sc_aggregate(vlo, vhi, p16, src_arr, tgt_arr):
    mesh = plsc.VectorSubcoreMesh(core_axis_name="c", subcore_axis_name="s")
    ZR = 128
    EPT = E // 16
    NCH2 = EPT // C

    @functools.partial(
        pl.kernel,
        out_type=(
            jax.ShapeDtypeStruct((2, NPAD, 64), jnp.float32),
            jax.ShapeDtypeStruct((2, NPAD, 16), jnp.float32),
        ),
        mesh=mesh,
        scratch_types=[
            pltpu.VMEM((C,), jnp.int32),
            pltpu.VMEM((C,), jnp.int32),
            pltpu.VMEM((C, 64), jnp.float32),
            pltpu.VMEM((C, 16), jnp.float32),
            pltpu.VMEM((ZR, 64), jnp.float32),
            pltpu.VMEM((NPT, 16), jnp.float32),
            pltpu.VMEM_SHARED((NPAD, 64), jnp.float32),
            pltpu.VMEM_SHARED((NPAD, 16), jnp.float32),
            pltpu.SemaphoreType.DMA,
        ],
        compiler_params=pltpu.CompilerParams(use_tc_tiling_on_sc=False),
    )
    def k(vlo_hbm, vhi_hbm, p_hbm, src_hbm, tgt_hbm, acc_out, z_out,
          tgt_v, src_v, vbuf, pbuf, zbuf, zbuf2, acc_sh, z_sh, sem):
        cid = lax.axis_index("c")
        sid = lax.axis_index("s")

        zero16 = jnp.zeros((16,), jnp.float32)

        def zrow(i, carry):
            for j in range(64 // 16):
                zbuf[i, pl.ds(j * 16, 16)] = zero16
            return carry

        lax.fori_loop(0, ZR, zrow, 0)

        def zrow2(i, carry):
            zbuf2[i, pl.ds(0, 16)] = zero16
            return carry

        lax.fori_loop(0, NPT, zrow2, 0)

        for kk in range(5):
            pltpu.sync_copy(zbuf, acc_sh.at[pl.ds(sid * NPT + kk * ZR, ZR)])
        pltpu.sync_copy(zbuf2, z_sh.at[pl.ds(sid * NPT, NPT)])
        plsc.subcore_barrier()

        hbase = 4 * cid

        def chunk(g, carry):
            base = sid * EPT + g * C
            pltpu.sync_copy(tgt_hbm.at[pl.ds(base, C)], tgt_v)
            pltpu.sync_copy(src_hbm.at[pl.ds(base, C)], src_v)

            @pl.when(cid == 0)
            def _():
                pltpu.async_copy(vlo_hbm.at[src_v], vbuf, sem).wait()

            @pl.when(cid == 1)
            def _():
                pltpu.async_copy(vhi_hbm.at[src_v], vbuf, sem).wait()

            pltpu.sync_copy(p_hbm.at[pl.ds(base, C)], pbuf)

            dn = lax.GatherDimensionNumbers(
                offset_dims=(), collapsed_slice_dims=(0,), start_index_map=(0,))

            def row(i, c2):
                pv = pbuf[i, pl.ds(0, 16)]
                for h in range(4):
                    idx = jnp.full((16, 1), hbase + h, jnp.int32)
                    ph = lax.gather(pv, idx, dn, (1,),
                                    mode=lax.GatherScatterMode.PROMISE_IN_BOUNDS)
                    sl = pl.ds(h * 16, 16)
                    vbuf[i, sl] = vbuf[i, sl] * ph
                return c2

            lax.fori_loop(0, C, row, 0)
            pltpu.sync_copy(vbuf, acc_sh.at[tgt_v], add=True)
            pltpu.sync_copy(pbuf, z_sh.at[tgt_v], add=True)
            return carry

        lax.fori_loop(0, NCH2, chunk, 0)
        plsc.subcore_barrier()

        pltpu.sync_copy(acc_sh.at[pl.ds(sid * NPT, NPT)],
                        acc_out.at[cid, pl.ds(sid * NPT, NPT)])
        pltpu.sync_copy(z_sh.at[pl.ds(sid * NPT, NPT)],
                        z_out.at[cid, pl.ds(sid * NPT, NPT)])

    return k(vlo, vhi, p16, src_arr, tgt_arr)


def _tc_final(acc, z, R2lo, R2hi, Wo, bo):
    BN = 1000

    def body(a_ref, z_ref, rlo_ref, rhi_ref, wlo_ref, whi_ref, b_ref, out):
        zs = z_ref[0] + z_ref[1]
        zinv = 1.0 / (zs + 1e-30)
        zrlo = jnp.dot(zinv, rlo_ref[...], preferred_element_type=jnp.float32)
        zrhi = jnp.dot(zinv, rhi_ref[...], preferred_element_type=jnp.float32)
        out[...] = (jnp.dot(a_ref[0] * zrlo, wlo_ref[...],
                            preferred_element_type=jnp.float32)
                    + jnp.dot(a_ref[1] * zrhi, whi_ref[...],
                              preferred_element_type=jnp.float32)
                    + b_ref[...])

    return pl.pallas_call(
        body,
        grid=(N // BN,),
        in_specs=[
            pl.BlockSpec((2, BN, 64), lambda i: (0, i, 0)),
            pl.BlockSpec((2, BN, 16), lambda i: (0, i, 0)),
            pl.BlockSpec((16, 64), lambda i: (0, 0)),
            pl.BlockSpec((16, 64), lambda i: (0, 0)),
            pl.BlockSpec((64, DF), lambda i: (0, 0)),
            pl.BlockSpec((64, DF), lambda i: (0, 0)),
            pl.BlockSpec((1, DF), lambda i: (0, 0)),
        ],
        out_specs=pl.BlockSpec((BN, DF), lambda i: (i, 0)),
        out_shape=jax.ShapeDtypeStruct((N, DF), jnp.float32),
    )(acc, z, R2lo, R2hi, Wo[:64], Wo[64:], bo.reshape(1, DF))


def kernel(node_features, edge_index, edge_features, Wl, bl, Wr, br, We, be,
           attn_vec, Wv, bv, Wo, bo):
    left, right, vlo, vhi = _tc_proj(node_features, Wl, bl, Wr, br, Wv, bv)
    src_arr = edge_index[0]
    tgt_arr = edge_index[1]
    combined = _sc_combine(left, right, src_arr, tgt_arr)
    A16 = jnp.repeat(jnp.eye(H, 16, dtype=jnp.float32), HD, axis=0) \
        * attn_vec.reshape(DF, 1)
    We16 = jnp.pad(We, ((0, 0), (0, 8)))
    be16 = jnp.pad(be, (0, 8)).reshape(1, 16)
    p16 = _tc_scores(combined, edge_features, A16, We16, be16)
    acc, z = _sc_aggregate(vlo, vhi, p16, src_arr, tgt_arr)
    R2 = 2.0 * jnp.repeat(jnp.eye(16, H, dtype=jnp.float32), HD, axis=1)
    return _tc_final(acc, z, R2[:, :64], R2[:, 64:], Wo, bo)


# speedup vs baseline: 10.3087x; 1.6056x over previous
import functools

import jax
import jax.numpy as jnp
from jax import lax
from jax.experimental import pallas as pl
from jax.experimental.pallas import tpu as pltpu
from jax.experimental.pallas import tpu_sc as plsc

N = 10000
E = 320000
DF = 128
H = 8
HD = 16
NW = 32
EPW = E // NW
C = 80
NCH = EPW // C
NPAD = 10240
NPT = NPAD // 16


def _tc_proj(x, Wl, bl, Wr, br, Wv, bv):
    BN = 1000

    def body(x_ref, wl, bl_, wr, br_, wv, bv_, lo, ro, vlo, vhi):
        xb = x_ref[...]
        lo[...] = jnp.dot(xb, wl[...], preferred_element_type=jnp.float32) + bl_[...]
        ro[...] = jnp.dot(xb, wr[...], preferred_element_type=jnp.float32) + br_[...]
        v = jnp.dot(xb, wv[...], preferred_element_type=jnp.float32) + bv_[...]
        vlo[...] = v[:, :64]
        vhi[...] = v[:, 64:]

    w_spec = pl.BlockSpec((DF, DF), lambda i: (0, 0))
    b_spec = pl.BlockSpec((1, DF), lambda i: (0, 0))
    o_spec = pl.BlockSpec((BN, DF), lambda i: (i, 0))
    h_spec = pl.BlockSpec((BN, 64), lambda i: (i, 0))
    return pl.pallas_call(
        body,
        grid=(N // BN,),
        in_specs=[o_spec, w_spec, b_spec, w_spec, b_spec, w_spec, b_spec],
        out_specs=[o_spec, o_spec, h_spec, h_spec],
        out_shape=[jax.ShapeDtypeStruct((N, DF), jnp.float32)] * 2
        + [jax.ShapeDtypeStruct((N, 64), jnp.float32)] * 2,
    )(x, Wl, bl.reshape(1, DF), Wr, br.reshape(1, DF), Wv, bv.reshape(1, DF))


def _sc_combine(left, right, src_arr, tgt_arr):
    mesh = plsc.VectorSubcoreMesh(core_axis_name="c", subcore_axis_name="s")
    CH = 128
    TOT = E // CH
    TOTW = (TOT + NW - 1) // NW
    NB = 3

    @functools.partial(
        pl.kernel,
        out_type=jax.ShapeDtypeStruct((E, DF), jnp.float32),
        mesh=mesh,
        scratch_types=[
            pltpu.VMEM((NB, CH), jnp.int32),
            pltpu.VMEM((NB, CH), jnp.int32),
            pltpu.VMEM((NB, CH, DF), jnp.float32),
            pltpu.VMEM((NB, CH, DF), jnp.float32),
            [pltpu.SemaphoreType.DMA] * NB,
            [pltpu.SemaphoreType.DMA] * NB,
        ],
    )
    def k(left_hbm, right_hbm, src_hbm, tgt_hbm, out_hbm,
          tgtb, srcb, lb, rb, semg, semo):
        wid = lax.axis_index("s") * 2 + lax.axis_index("c")

        def cbase(g):
            return jnp.minimum(wid + g * NW, TOT - 1) * CH

        def fire(g, p):
            base = cbase(g)
            pltpu.sync_copy(tgt_hbm.at[pl.ds(base, CH)], tgtb.at[p])
            pltpu.sync_copy(src_hbm.at[pl.ds(base, CH)], srcb.at[p])
            pltpu.async_copy(left_hbm.at[tgtb.at[p]], lb.at[p], semg[p])
            pltpu.async_copy(right_hbm.at[srcb.at[p]], rb.at[p], semg[p])

        def wait_out(p):
            pltpu.make_async_copy(lb.at[p], out_hbm.at[pl.ds(0, CH)],
                                  semo[p]).wait()

        def proc(g, p):
            pltpu.make_async_copy(left_hbm.at[pl.ds(0, CH)], lb.at[p],
                                  semg[p]).wait()
            pltpu.make_async_copy(left_hbm.at[pl.ds(0, CH)], rb.at[p],
                                  semg[p]).wait()

            def row(i, c2):
                for j in range(DF // 16):
                    sl = pl.ds(j * 16, 16)
                    lb[p, i, sl] = lb[p, i, sl] + rb[p, i, sl]
                return c2

            lax.fori_loop(0, CH, row, 0)
            pltpu.async_copy(lb.at[p], out_hbm.at[pl.ds(cbase(g), CH)], semo[p])

        for p in range(NB):
            fire(p, p)

        def body(g, carry):
            par = lax.rem(g, NB)
            for p in range(NB):
                @pl.when(par == p)
                def _(p=p, g=g):
                    proc(g, p)

                    @pl.when((par == p) & (g + NB < TOTW))
                    def _(p=p, g=g):
                        wait_out(p)
                        fire(g + NB, p)
            return carry

        lax.fori_loop(0, TOTW, body, 0)
        for p in range(NB):
            wait_out(p)

    return k(left, right, src_arr, tgt_arr)


def _tc_scores(combined, ef, A16, We16, be16):
    BE = 2000

    def body(c_ref, e_ref, a_ref, w_ref, b_ref, out):
        c = c_ref[...]
        act = jnp.where(c >= 0, c, 0.2 * c)
        s = (jnp.dot(act, a_ref[...], preferred_element_type=jnp.float32)
             + jnp.dot(e_ref[...], w_ref[...], preferred_element_type=jnp.float32)
             + b_ref[...])
        out[...] = jnp.exp(s)

    return pl.pallas_call(
        body,
        grid=(E // BE,),
        in_specs=[
            pl.BlockSpec((BE, DF), lambda i: (i, 0)),
            pl.BlockSpec((BE, 16), lambda i: (i, 0)),
            pl.BlockSpec((DF, 16), lambda i: (0, 0)),
            pl.BlockSpec((16, 16), lambda i: (0, 0)),
            pl.BlockSpec((1, 16), lambda i: (0, 0)),
        ],
        out_specs=pl.BlockSpec((BE, 16), lambda i: (i, 0)),
        out_shape=jax.ShapeDtypeStruct((E, 16), jnp.float32),
    )(combined, ef, A16, We16, be16)


def _sc_aggregate(vlo, vhi, p16, src_arr, tgt_arr):
    mesh = plsc.VectorSubcoreMesh(core_axis_name="c", subcore_axis_name="s")
    ZR = 128
    CH = 128
    NB = 3
    TOT = E // CH
    NCHT = TOT // 16
    REMT = TOT - NCHT * 16

    @functools.partial(
        pl.kernel,
        out_type=(
            jax.ShapeDtypeStruct((2, NPAD, 64), jnp.float32),
            jax.ShapeDtypeStruct((2, NPAD, 16), jnp.float32),
        ),
        mesh=mesh,
        scratch_types=[
            pltpu.VMEM((NB, CH), jnp.int32),
            pltpu.VMEM((NB, CH), jnp.int32),
            pltpu.VMEM((NB, CH, 64), jnp.float32),
            pltpu.VMEM((NB, CH, 16), jnp.float32),
            pltpu.VMEM((ZR, 64), jnp.float32),
            pltpu.VMEM((NPT, 16), jnp.float32),
            pltpu.VMEM_SHARED((NPAD, 64), jnp.float32),
            pltpu.VMEM_SHARED((NPAD, 16), jnp.float32),
            [pltpu.SemaphoreType.DMA] * NB,
            [pltpu.SemaphoreType.DMA] * NB,
        ],
        compiler_params=pltpu.CompilerParams(use_tc_tiling_on_sc=False),
    )
    def k(vlo_hbm, vhi_hbm, p_hbm, src_hbm, tgt_hbm, acc_out, z_out,
          tgtb, srcb, vb, pb, zbuf, zbuf2, acc_sh, z_sh, semg, semo):
        cid = lax.axis_index("c")
        sid = lax.axis_index("s")

        zero16 = jnp.zeros((16,), jnp.float32)

        def zrow(i, carry):
            for j in range(64 // 16):
                zbuf[i, pl.ds(j * 16, 16)] = zero16
            return carry

        lax.fori_loop(0, ZR, zrow, 0)

        def zrow2(i, carry):
            zbuf2[i, pl.ds(0, 16)] = zero16
            return carry

        lax.fori_loop(0, NPT, zrow2, 0)

        for kk in range(5):
            pltpu.sync_copy(zbuf, acc_sh.at[pl.ds(sid * NPT + kk * ZR, ZR)])
        pltpu.sync_copy(zbuf2, z_sh.at[pl.ds(sid * NPT, NPT)])
        plsc.subcore_barrier()

        hbase = 4 * cid
        dn = lax.GatherDimensionNumbers(
            offset_dims=(), collapsed_slice_dims=(0,), start_index_map=(0,))

        def cbase(g):
            return (sid + 16 * g) * CH

        def fire(g, p):
            base = cbase(g)
            pltpu.sync_copy(tgt_hbm.at[pl.ds(base, CH)], tgtb.at[p])
            pltpu.sync_copy(src_hbm.at[pl.ds(base, CH)], srcb.at[p])

            @pl.when(cid == 0)
            def _():
                pltpu.async_copy(vlo_hbm.at[srcb.at[p]], vb.at[p], semg[p])

            @pl.when(cid == 1)
            def _():
                pltpu.async_copy(vhi_hbm.at[srcb.at[p]], vb.at[p], semg[p])

            pltpu.async_copy(p_hbm.at[pl.ds(base, CH)], pb.at[p], semg[p])

        def wait_out(p):
            pltpu.make_async_copy(vb.at[p], acc_sh.at[pl.ds(0, CH)],
                                  semo[p]).wait()
            pltpu.make_async_copy(pb.at[p], z_sh.at[pl.ds(0, CH)],
                                  semo[p]).wait()

        def proc(p):
            pltpu.make_async_copy(vlo_hbm.at[pl.ds(0, CH)], vb.at[p],
                                  semg[p]).wait()
            pltpu.make_async_copy(p_hbm.at[pl.ds(0, CH)], pb.at[p],
                                  semg[p]).wait()

            def row(i, c2):
                pv = pb[p, i, pl.ds(0, 16)]
                for h in range(4):
                    idx = jnp.full((16, 1), hbase + h, jnp.int32)
                    ph = lax.gather(pv, idx, dn, (1,),
                                    mode=lax.GatherScatterMode.PROMISE_IN_BOUNDS)
                    sl = pl.ds(h * 16, 16)
                    vb[p, i, sl] = vb[p, i, sl] * ph
                return c2

            lax.fori_loop(0, CH, row, 0)
            pltpu.async_copy(vb.at[p], acc_sh.at[tgtb.at[p]], semo[p], add=True)
            pltpu.async_copy(pb.at[p], z_sh.at[tgtb.at[p]], semo[p], add=True)

        nch = NCHT + jnp.where(sid < REMT, 1, 0)

        for p in range(NB):
            @pl.when(p < nch)
            def _(p=p):
                fire(p, p)

        def body(g, carry):
            par = lax.rem(g, NB)
            for p in range(NB):
                @pl.when(par == p)
                def _(p=p, g=g):
                    proc(p)

                    @pl.when((par == p) & (g + NB < nch))
                    def _(p=p, g=g):
                        wait_out(p)
                        fire(g + NB, p)
            return carry

        lax.fori_loop(0, nch, body, 0)
        for p in range(NB):
            @pl.when(p < nch)
            def _(p=p):
                wait_out(p)
        plsc.subcore_barrier()

        pltpu.sync_copy(acc_sh.at[pl.ds(sid * NPT, NPT)],
                        acc_out.at[cid, pl.ds(sid * NPT, NPT)])
        pltpu.sync_copy(z_sh.at[pl.ds(sid * NPT, NPT)],
                        z_out.at[cid, pl.ds(sid * NPT, NPT)])

    return k(vlo, vhi, p16, src_arr, tgt_arr)


def _tc_final(acc, z, R2lo, R2hi, Wo, bo):
    BN = 1000

    def body(a_ref, z_ref, rlo_ref, rhi_ref, wlo_ref, whi_ref, b_ref, out):
        zs = z_ref[0] + z_ref[1]
        zinv = 1.0 / (zs + 1e-30)
        zrlo = jnp.dot(zinv, rlo_ref[...], preferred_element_type=jnp.float32)
        zrhi = jnp.dot(zinv, rhi_ref[...], preferred_element_type=jnp.float32)
        out[...] = (jnp.dot(a_ref[0] * zrlo, wlo_ref[...],
                            preferred_element_type=jnp.float32)
                    + jnp.dot(a_ref[1] * zrhi, whi_ref[...],
                              preferred_element_type=jnp.float32)
                    + b_ref[...])

    return pl.pallas_call(
        body,
        grid=(N // BN,),
        in_specs=[
            pl.BlockSpec((2, BN, 64), lambda i: (0, i, 0)),
            pl.BlockSpec((2, BN, 16), lambda i: (0, i, 0)),
            pl.BlockSpec((16, 64), lambda i: (0, 0)),
            pl.BlockSpec((16, 64), lambda i: (0, 0)),
            pl.BlockSpec((64, DF), lambda i: (0, 0)),
            pl.BlockSpec((64, DF), lambda i: (0, 0)),
            pl.BlockSpec((1, DF), lambda i: (0, 0)),
        ],
        out_specs=pl.BlockSpec((BN, DF), lambda i: (i, 0)),
        out_shape=jax.ShapeDtypeStruct((N, DF), jnp.float32),
    )(acc, z, R2lo, R2hi, Wo[:64], Wo[64:], bo.reshape(1, DF))


def kernel(node_features, edge_index, edge_features, Wl, bl, Wr, br, We, be,
           attn_vec, Wv, bv, Wo, bo):
    left, right, vlo, vhi = _tc_proj(node_features, Wl, bl, Wr, br, Wv, bv)
    src_arr = edge_index[0]
    tgt_arr = edge_index[1]
    combined = _sc_combine(left, right, src_arr, tgt_arr)
    A16 = jnp.repeat(jnp.eye(H, 16, dtype=jnp.float32), HD, axis=0) \
        * attn_vec.reshape(DF, 1)
    We16 = jnp.pad(We, ((0, 0), (0, 8)))
    be16 = jnp.pad(be, (0, 8)).reshape(1, 16)
    p16 = _tc_scores(combined, edge_features, A16, We16, be16)
    acc, z = _sc_aggregate(vlo, vhi, p16, src_arr, tgt_arr)
    R2 = 2.0 * jnp.repeat(jnp.eye(16, H, dtype=jnp.float32), HD, axis=1)
    return _tc_final(acc, z, R2[:, :64], R2[:, 64:], Wo, bo)
